# baseline (speedup 1.0000x reference)
import functools

import jax
import jax.numpy as jnp
from jax import lax
from jax.experimental import pallas as pl
from jax.experimental.pallas import tpu as pltpu
from jax.experimental.pallas import tpu_sc as plsc

VOCAB = 30522
HIDDEN = 1024
MAX_POS = 512
BATCH = 64
SEQ = 512
EPS = 1e-12

L = 16
NC = 2
NS = 16
NW = NC * NS
PW = SEQ // NW
HC = HIDDEN // L
TG = 8
KA = 8


def _rsqrt_vec(x):
    i = plsc.bitcast(x, jnp.int32)
    i = jnp.int32(0x5F3759DF) - (i >> 1)
    y = plsc.bitcast(i, jnp.float32)
    half = x * 0.5
    for _ in range(4):
        y = y * (1.5 - half * y * y)
    return y


NB = 4


def _body(ids_h, tt_h, we_h, pe_h, te_h, g_h, b_h, out_h,
          pe2, gb, idxall, ttall, wrows,
          gsem0, gsem1, gsem2, gsem3, osem0, osem1, osem2, osem3):
    cid = lax.axis_index("c")
    sid = lax.axis_index("s")
    wid = sid * NC + cid
    base_s = wid * PW
    gsem = (gsem0, gsem1, gsem2, gsem3)
    osem = (osem0, osem1, osem2, osem3)

    pltpu.sync_copy(ids_h.at[wid], idxall)
    pltpu.sync_copy(tt_h.at[wid], ttall)
    pltpu.sync_copy(pe_h.at[pl.ds(base_s, PW)], wrows.at[0])
    pltpu.sync_copy(g_h, gb.at[0])
    pltpu.sync_copy(b_h, gb.at[1])
    pltpu.sync_copy(te_h, gb.at[pl.ds(2, 2)])

    def fold_te(h):
        off = h * L
        tes = [gb[2 + tt, pl.ds(off, L)] for tt in range(2)]
        for p in range(PW):
            a = wrows[0, p, pl.ds(off, L)]
            for tt in range(2):
                pe2[tt * PW + p, pl.ds(off, L)] = a + tes[tt]
    plsc.parallel_loop(0, HC, 1)(fold_te)

    pltpu.async_copy(we_h.at[idxall.at[pl.ds(0, PW)]], wrows.at[0], gsem[0])
    pltpu.async_copy(we_h.at[idxall.at[pl.ds(PW, PW)]], wrows.at[1], gsem[1])

    def batch_quad(j, _):
        for buf in range(NB):
            b = NB * j + buf
            nbuf = (buf + 2) % NB

            @pl.when(b + 2 < BATCH)
            def _():
                @pl.when(b >= 2)
                def _():
                    pltpu.make_async_copy(
                        wrows.at[nbuf],
                        out_h.at[b - 2, pl.ds(base_s, PW)],
                        osem[nbuf]).wait()
                pltpu.async_copy(we_h.at[idxall.at[pl.ds((b + 2) * PW, PW)]],
                                 wrows.at[nbuf], gsem[nbuf])

            pltpu.make_async_copy(we_h.at[idxall.at[pl.ds(b * PW, PW)]],
                                  wrows.at[buf], gsem[buf]).wait()

            ttv = (ttall[pl.ds(b * PW, PW)] * PW + lax.iota(jnp.int32, L))
            zero = jnp.zeros((L,), jnp.float32)
            lane = lax.iota(jnp.int32, L)

            rows = [ttv[p] for p in range(PW)]
            s1v = zero
            s2v = zero
            QT = 8

            for q in range(PW // QT):
                ps = [q * QT + i for i in range(QT)]

                def pass1(h, carry):
                    accs, asqs = carry
                    accs, asqs = list(accs), list(asqs)
                    off = h * L
                    for i, p in enumerate(ps):
                        e = (wrows[buf, p, pl.ds(off, L)]
                             + pe2[rows[p], pl.ds(off, L)])
                        wrows[buf, p, pl.ds(off, L)] = e
                        accs[i] = accs[i] + e
                        asqs[i] = asqs[i] + e * e
                    return tuple(accs), tuple(asqs)

                carry0 = ((zero,) * QT, (zero,) * QT)
                accs, asqs = plsc.parallel_loop(0, HC, 1,
                                                carry=carry0)(pass1)
                for i, p in enumerate(ps):
                    s1v = jnp.where(lane == p, jnp.sum(accs[i]), s1v)
                    s2v = jnp.where(lane == p, jnp.sum(asqs[i]), s2v)

            mean_v = s1v * (1.0 / HIDDEN)
            var_v = s2v * (1.0 / HIDDEN) - mean_v * mean_v
            rinv_v = _rsqrt_vec(var_v + EPS)
            shift_v = -mean_v * rinv_v
            scale_shift = [(jnp.full((L,), rinv_v[p]),
                            jnp.full((L,), shift_v[p])) for p in range(PW)]

            for g in range(PW // TG):
                def pass2(h):
                    off = h * L
                    gg = gb[0, pl.ds(off, L)]
                    bb = gb[1, pl.ds(off, L)]
                    for p8 in range(TG):
                        p = g * TG + p8
                        a_p, b_p = scale_shift[p]
                        e = wrows[buf, p, pl.ds(off, L)]
                        wrows[buf, p, pl.ds(off, L)] = (e * a_p + b_p) * gg + bb
                plsc.parallel_loop(0, HC, 1, unroll=4)(pass2)

            pltpu.async_copy(wrows.at[buf],
                             out_h.at[b, pl.ds(base_s, PW)], osem[buf])
        return 0

    lax.fori_loop(0, BATCH // NB, batch_quad, 0)

    for k in range(NB):
        bq = BATCH - NB + k
        pltpu.make_async_copy(wrows.at[k],
                              out_h.at[bq, pl.ds(base_s, PW)],
                              osem[k]).wait()


@jax.jit
def _sc_embed(ids_t, tt_t, word_embeddings,
              position_embeddings, token_type_embeddings, ln_gamma, ln_beta):
    mesh = plsc.VectorSubcoreMesh(core_axis_name="c", subcore_axis_name="s",
                                  num_cores=NC, num_subcores=NS)
    f = pl.kernel(
        _body,
        out_type=jax.ShapeDtypeStruct((BATCH, SEQ, HIDDEN), jnp.float32),
        mesh=mesh,
        compiler_params=pltpu.CompilerParams(needs_layout_passes=False),
        scratch_types=[
            pltpu.VMEM((2 * PW, HIDDEN), jnp.float32),
            pltpu.VMEM((4, HIDDEN), jnp.float32),
            pltpu.VMEM((BATCH * PW,), jnp.int32),
            pltpu.VMEM((BATCH * PW,), jnp.int32),
            pltpu.VMEM((NB, PW, HIDDEN), jnp.float32),
            pltpu.SemaphoreType.DMA,
            pltpu.SemaphoreType.DMA,
            pltpu.SemaphoreType.DMA,
            pltpu.SemaphoreType.DMA,
            pltpu.SemaphoreType.DMA,
            pltpu.SemaphoreType.DMA,
            pltpu.SemaphoreType.DMA,
            pltpu.SemaphoreType.DMA,
        ],
    )
    return f(ids_t, tt_t, word_embeddings,
             position_embeddings, token_type_embeddings, ln_gamma, ln_beta)


def kernel(input_ids, token_type_ids, word_embeddings, position_embeddings,
           token_type_embeddings, ln_gamma, ln_beta):
    ids_t = (input_ids.reshape(BATCH, NW, PW).transpose(1, 0, 2)
             .reshape(NW, BATCH * PW))
    tt_t = (token_type_ids.reshape(BATCH, NW, PW).transpose(1, 0, 2)
            .reshape(NW, BATCH * PW))
    return _sc_embed(ids_t, tt_t, word_embeddings,
                     position_embeddings, token_type_embeddings,
                     ln_gamma, ln_beta)


# speedup vs baseline: 1.3794x; 1.3794x over previous
import functools

import jax
import jax.numpy as jnp
from jax import lax
from jax.experimental import pallas as pl
from jax.experimental.pallas import tpu as pltpu
from jax.experimental.pallas import tpu_sc as plsc

VOCAB = 30522
HIDDEN = 1024
MAX_POS = 512
BATCH = 64
SEQ = 512
EPS = 1e-12

L = 16
NC = 2
NS = 16
NW = NC * NS
PW = SEQ // NW
HC = HIDDEN // L
TG = 8
QT = 8


def _rsqrt_vec(x):
    i = plsc.bitcast(x, jnp.int32)
    i = jnp.int32(0x5F3759DF) - (i >> 1)
    y = plsc.bitcast(i, jnp.float32)
    half = x * 0.5
    for _ in range(4):
        y = y * (1.5 - half * y * y)
    return y


def _body(ids_h, tt_h, we_h, pe_h, te_h, g_h, b_h, out_h,
          pe2, gb, idxall, ttall, wrows, obuf,
          gsem0, gsem1, osem0, osem1):
    cid = lax.axis_index("c")
    sid = lax.axis_index("s")
    wid = sid * NC + cid
    base_s = wid * PW
    gsem = (gsem0, gsem1)
    osem = (osem0, osem1)

    pltpu.sync_copy(ids_h.at[wid], idxall)
    pltpu.sync_copy(tt_h.at[wid], ttall)
    pltpu.sync_copy(pe_h.at[pl.ds(base_s, PW)], wrows.at[0])
    pltpu.sync_copy(g_h, gb.at[0])
    pltpu.sync_copy(b_h, gb.at[1])
    pltpu.sync_copy(te_h, gb.at[pl.ds(2, 2)])

    def fold_te(h):
        off = h * L
        tes = [gb[2 + tt, pl.ds(off, L)] for tt in range(2)]
        for p in range(PW):
            a = wrows[0, p, pl.ds(off, L)]
            for tt in range(2):
                pe2[tt * PW + p, pl.ds(off, L)] = a + tes[tt]
    plsc.parallel_loop(0, HC, 1)(fold_te)

    pltpu.async_copy(we_h.at[idxall.at[pl.ds(0, PW)]], wrows.at[0], gsem[0])

    def batch_pair(j, _):
        for buf in (0, 1):
            b = 2 * j + buf
            nbuf = 1 - buf

            @pl.when(b < BATCH - 1)
            def _():
                pltpu.async_copy(we_h.at[idxall.at[pl.ds((b + 1) * PW, PW)]],
                                 wrows.at[nbuf], gsem[nbuf])

            pltpu.make_async_copy(we_h.at[idxall.at[pl.ds(b * PW, PW)]],
                                  wrows.at[buf], gsem[buf]).wait()

            ttv = (ttall[pl.ds(b * PW, PW)] * PW + lax.iota(jnp.int32, L))
            zero = jnp.zeros((L,), jnp.float32)
            lane = lax.iota(jnp.int32, L)

            rows = [ttv[p] for p in range(PW)]
            s1v = zero
            s2v = zero

            for q in range(PW // QT):
                ps = [q * QT + i for i in range(QT)]

                def pass1(h, carry):
                    accs, asqs = carry
                    accs, asqs = list(accs), list(asqs)
                    off = h * L
                    for i, p in enumerate(ps):
                        e = (wrows[buf, p, pl.ds(off, L)]
                             + pe2[rows[p], pl.ds(off, L)])
                        wrows[buf, p, pl.ds(off, L)] = e
                        accs[i] = accs[i] + e
                        asqs[i] = asqs[i] + e * e
                    return tuple(accs), tuple(asqs)

                carry0 = ((zero,) * QT, (zero,) * QT)
                accs, asqs = plsc.parallel_loop(0, HC, 1,
                                                carry=carry0)(pass1)
                for i, p in enumerate(ps):
                    s1v = jnp.where(lane == p, jnp.sum(accs[i]), s1v)
                    s2v = jnp.where(lane == p, jnp.sum(asqs[i]), s2v)

            mean_v = s1v * (1.0 / HIDDEN)
            var_v = s2v * (1.0 / HIDDEN) - mean_v * mean_v
            rinv_v = _rsqrt_vec(var_v + EPS)
            shift_v = -mean_v * rinv_v
            scale_shift = [(jnp.full((L,), rinv_v[p]),
                            jnp.full((L,), shift_v[p])) for p in range(PW)]

            for g in range(PW // TG):
                def pass2(h):
                    off = h * L
                    gg = gb[0, pl.ds(off, L)]
                    bb = gb[1, pl.ds(off, L)]
                    for p8 in range(TG):
                        p = g * TG + p8
                        a_p, b_p = scale_shift[p]
                        e = wrows[buf, p, pl.ds(off, L)]
                        obuf[buf, p, pl.ds(off, L)] = (e * a_p + b_p) * gg + bb
                plsc.parallel_loop(0, HC, 1, unroll=4)(pass2)

            pltpu.async_copy(obuf.at[buf],
                             out_h.at[b, pl.ds(base_s, PW)], osem[buf])
        return 0

    lax.fori_loop(0, BATCH // 2, batch_pair, 0)

    pltpu.make_async_copy(obuf.at[0], out_h.at[BATCH - 2, pl.ds(base_s, PW)],
                          osem[0]).wait()
    pltpu.make_async_copy(obuf.at[1], out_h.at[BATCH - 1, pl.ds(base_s, PW)],
                          osem[1]).wait()


@jax.jit
def _sc_embed(ids_t, tt_t, word_embeddings,
              position_embeddings, token_type_embeddings, ln_gamma, ln_beta):
    mesh = plsc.VectorSubcoreMesh(core_axis_name="c", subcore_axis_name="s",
                                  num_cores=NC, num_subcores=NS)
    f = pl.kernel(
        _body,
        out_type=jax.ShapeDtypeStruct((BATCH, SEQ, HIDDEN), jnp.float32),
        mesh=mesh,
        compiler_params=pltpu.CompilerParams(needs_layout_passes=False),
        scratch_types=[
            pltpu.VMEM((2 * PW, HIDDEN), jnp.float32),
            pltpu.VMEM((4, HIDDEN), jnp.float32),
            pltpu.VMEM((BATCH * PW,), jnp.int32),
            pltpu.VMEM((BATCH * PW,), jnp.int32),
            pltpu.VMEM((2, PW, HIDDEN), jnp.float32),
            pltpu.VMEM((2, PW, HIDDEN), jnp.float32),
            pltpu.SemaphoreType.DMA,
            pltpu.SemaphoreType.DMA,
            pltpu.SemaphoreType.DMA,
            pltpu.SemaphoreType.DMA,
        ],
    )
    return f(ids_t, tt_t, word_embeddings,
             position_embeddings, token_type_embeddings, ln_gamma, ln_beta)


def kernel(input_ids, token_type_ids, word_embeddings, position_embeddings,
           token_type_embeddings, ln_gamma, ln_beta):
    ids_t = (input_ids.reshape(BATCH, NW, PW).transpose(1, 0, 2)
             .reshape(NW, BATCH * PW))
    tt_t = (token_type_ids.reshape(BATCH, NW, PW).transpose(1, 0, 2)
            .reshape(NW, BATCH * PW))
    return _sc_embed(ids_t, tt_t, word_embeddings,
                     position_embeddings, token_type_embeddings,
                     ln_gamma, ln_beta)
